# baseline (speedup 1.0000x reference)
import functools

import jax
import jax.numpy as jnp
from jax import lax
from jax.experimental import pallas as pl
from jax.experimental.pallas import tpu as pltpu
from jax.experimental.pallas import tpu_sc as plsc

F32 = jnp.float32
BF16 = jnp.bfloat16
I32 = jnp.int32

NW = 32
LANE = 16
CHUNK = 128


def _sc_mesh():
    return plsc.VectorSubcoreMesh(core_axis_name="c", subcore_axis_name="s")


def _sc_degree(dst_r, w_r, np_, ce):
    rows_per_tile = np_ // 16

    def body(dst_hbm, w_hbm, out_hbm, dstv, wv, zbuf, acc, _sem):
        cid = lax.axis_index("c")
        sid = lax.axis_index("s")
        wid = sid * 2 + cid
        pltpu.sync_copy(dst_hbm.at[wid], dstv)
        pltpu.sync_copy(w_hbm.at[wid], wv)

        @pl.loop(0, CHUNK // LANE)
        def _zero(j):
            zbuf[pl.ds(j * LANE, LANE)] = jnp.zeros((LANE,), F32)

        @pl.loop(0, rows_per_tile // CHUNK)
        def _zacc(j):
            pltpu.sync_copy(zbuf, acc.at[pl.ds(sid * rows_per_tile + j * CHUNK, CHUNK)])

        plsc.subcore_barrier()

        @pl.loop(0, ce)
        def _scatter(c):
            pltpu.sync_copy(wv.at[c], acc.at[dstv.at[c]], add=True)

        plsc.subcore_barrier()
        pltpu.sync_copy(acc.at[pl.ds(sid * rows_per_tile, rows_per_tile)],
                        out_hbm.at[cid, pl.ds(sid * rows_per_tile, rows_per_tile)])

    k = pl.kernel(
        body,
        out_type=jax.ShapeDtypeStruct((2, np_), F32),
        mesh=_sc_mesh(),
        compiler_params=pltpu.CompilerParams(needs_layout_passes=False),
        scratch_types=[
            pltpu.VMEM((ce, CHUNK), I32),
            pltpu.VMEM((ce, CHUNK), F32),
            pltpu.VMEM((CHUNK,), F32),
            pltpu.VMEM_SHARED((np_,), F32),
            pltpu.SemaphoreType.DMA,
        ],
    )
    return k(dst_r, w_r)


def _sc_messages(hb0, hb1, src_r, dst_r, w_r, np_, d, ce2):
    rows_per_tile = np_ // 16
    dh = d // 2
    hb0 = lax.bitcast_convert_type(hb0.reshape(np_, dh // 2, 2), F32)
    hb1 = lax.bitcast_convert_type(hb1.reshape(np_, dh // 2, 2), F32)

    def body(h0_hbm, h1_hbm, src_hbm, dst_hbm, w_hbm, out_hbm, srcv,
             rows0, rows1, srows, dstb0, dstb1, wb0, wb1,
             gsem0, gsem1, dsem0, dsem1, hsh, acc):
        cid = lax.axis_index("c")
        sid = lax.axis_index("s")
        wid = sid
        pltpu.sync_copy(src_hbm.at[wid], srcv)
        rsl = pl.ds(sid * rows_per_tile, rows_per_tile)

        @pl.when(cid == 0)
        def _():
            pltpu.sync_copy(h0_hbm.at[rsl], hsh.at[rsl])

        @pl.when(cid == 1)
        def _():
            pltpu.sync_copy(h1_hbm.at[rsl], hsh.at[rsl])

        @pl.loop(0, CHUNK)
        def _zrows(g):
            for j in range(dh // LANE):
                srows[g, pl.ds(j * LANE, LANE)] = jnp.zeros((LANE,), F32)

        @pl.loop(0, rows_per_tile // CHUNK)
        def _zacc(j):
            pltpu.sync_copy(srows, acc.at[pl.ds(sid * rows_per_tile + j * CHUNK, CHUNK)])

        plsc.subcore_barrier()

        def issue(c, rows, dstb, wb, gsem, dsem):
            pltpu.async_copy(hsh.at[srcv.at[c]], rows, gsem)
            pltpu.async_copy(dst_hbm.at[wid, c], dstb, dsem)
            pltpu.async_copy(w_hbm.at[wid, c], wb, dsem)

        def drain(c, rows, dstb, wb, gsem, dsem):
            pltpu.make_async_copy(hsh.at[srcv.at[c]], rows, gsem).wait()
            pltpu.make_async_copy(dst_hbm.at[wid, c], dstb, dsem).wait()
            pltpu.make_async_copy(w_hbm.at[wid, c], wb, dsem).wait()

        def scale(rows, wb):
            @pl.loop(0, CHUNK)
            def _scale(g):
                wbr = plsc.load_gather(wb, [jnp.full((LANE,), g, I32)])
                for j in range(dh // (2 * LANE)):
                    ab = plsc.bitcast(rows[g, pl.ds(LANE * j, LANE)], BF16)
                    a, b = plsc.unpack(ab, format=plsc.PackFormat.INTERLEAVED)
                    srows[g, pl.ds(2 * LANE * j, LANE)] = a * wbr
                    srows[g, pl.ds(2 * LANE * j + LANE, LANE)] = b * wbr

        issue(0, rows0, dstb0, wb0, gsem0, dsem0)
        issue(1, rows1, dstb1, wb1, gsem1, dsem1)

        @pl.loop(0, ce2 // 2)
        def _edge_pair(k):
            c0 = 2 * k
            drain(c0, rows0, dstb0, wb0, gsem0, dsem0)
            scale(rows0, wb0)
            pltpu.sync_copy(srows, acc.at[dstb0], add=True)

            @pl.when(k < ce2 // 2 - 1)
            def _():
                issue(c0 + 2, rows0, dstb0, wb0, gsem0, dsem0)

            drain(c0 + 1, rows1, dstb1, wb1, gsem1, dsem1)
            scale(rows1, wb1)
            pltpu.sync_copy(srows, acc.at[dstb1], add=True)

            @pl.when(k < ce2 // 2 - 1)
            def _():
                issue(c0 + 3, rows1, dstb1, wb1, gsem1, dsem1)

        plsc.subcore_barrier()
        pltpu.sync_copy(acc.at[pl.ds(sid * rows_per_tile, rows_per_tile)],
                        out_hbm.at[cid, pl.ds(sid * rows_per_tile, rows_per_tile)])

    k = pl.kernel(
        body,
        out_type=jax.ShapeDtypeStruct((2, np_, dh), F32),
        mesh=_sc_mesh(),
        compiler_params=pltpu.CompilerParams(needs_layout_passes=False,
                                             use_tc_tiling_on_sc=False),
        scratch_types=[
            pltpu.VMEM((ce2, CHUNK), I32),
            pltpu.VMEM((CHUNK, dh // 2), F32),
            pltpu.VMEM((CHUNK, dh // 2), F32),
            pltpu.VMEM((CHUNK, dh), F32),
            pltpu.VMEM((CHUNK,), I32),
            pltpu.VMEM((CHUNK,), I32),
            pltpu.VMEM((CHUNK,), F32),
            pltpu.VMEM((CHUNK,), F32),
            pltpu.SemaphoreType.DMA,
            pltpu.SemaphoreType.DMA,
            pltpu.SemaphoreType.DMA,
            pltpu.SemaphoreType.DMA,
            pltpu.VMEM_SHARED((np_, dh // 2), F32),
            pltpu.VMEM_SHARED((np_, dh), F32),
        ],
    )
    return k(hb0, hb1, src_r, dst_r, w_r)


def _tc_image_mean(images_r):
    rows, cols = images_r.shape
    blk = 1024
    steps = cols // blk

    def body(x_ref, out_ref, acc):
        i = pl.program_id(0)

        @pl.when(i == 0)
        def _():
            acc[...] = jnp.zeros_like(acc)

        x = x_ref[...]
        s = jnp.zeros((rows, 128), F32)
        for j in range(blk // 128):
            s = s + x[:, j * 128:(j + 1) * 128]
        acc[...] += s

        @pl.when(i == steps - 1)
        def _():
            out_ref[...] = acc[...]

    return pl.pallas_call(
        body,
        grid=(steps,),
        in_specs=[pl.BlockSpec((rows, blk), lambda i: (0, i))],
        out_specs=pl.BlockSpec((rows, 128), lambda i: (0, 0)),
        out_shape=jax.ShapeDtypeStruct((rows, 128), F32),
        scratch_shapes=[pltpu.VMEM((rows, 128), F32)],
    )(images_r)


def _tc_embed_conv1(xn_f, table_p, degb, w1, w1p, np_, d, blk):
    steps = np_ // blk

    def body(xn_ref, tbl_ref, deg_ref, w_ref, wp_ref, out_ref, outb0_ref, outb1_ref):
        xn_row = xn_ref[0]
        ioc = lax.broadcasted_iota(I32, (32, blk), 0).astype(F32)
        oht = (ioc == xn_row).astype(F32)
        xg = lax.dot_general(oht, tbl_ref[...],
                             (((0,), (0,)), ((), ())),
                             preferred_element_type=F32)
        dinv = lax.rsqrt(deg_ref[...] + 1.0)
        h = lax.dot_general(xg, w_ref[...], (((1,), (1,)), ((), ())),
                            preferred_element_type=F32)
        out_ref[...] = h * dinv
        hp = (lax.dot_general(xg, wp_ref[...], (((1,), (1,)), ((), ())),
                              preferred_element_type=F32) * dinv)
        outb0_ref[...] = hp[:, :d // 2].astype(BF16)
        outb1_ref[...] = hp[:, d // 2:].astype(BF16)

    return pl.pallas_call(
        body,
        grid=(steps,),
        in_specs=[
            pl.BlockSpec((1, 1, blk), lambda i: (i, 0, 0)),
            pl.BlockSpec((32, d), lambda i: (0, 0)),
            pl.BlockSpec((blk, d), lambda i: (i, 0)),
            pl.BlockSpec((d, d), lambda i: (0, 0)),
            pl.BlockSpec((d, d), lambda i: (0, 0)),
        ],
        out_specs=[pl.BlockSpec((blk, d), lambda i: (i, 0)),
                   pl.BlockSpec((blk, d // 2), lambda i: (i, 0)),
                   pl.BlockSpec((blk, d // 2), lambda i: (i, 0))],
        out_shape=[jax.ShapeDtypeStruct((np_, d), F32),
                   jax.ShapeDtypeStruct((np_, d // 2), BF16),
                   jax.ShapeDtypeStruct((np_, d // 2), BF16)],
    )(xn_f, table_p, degb, w1, w1p)


def _tc_conv_next(m, hp, degb, b_prev, w, wp, np_, d, blk):
    steps = np_ // blk

    def body(m_ref, hp_ref, deg_ref, b_ref, w_ref, wp_ref, out_ref, outb0_ref, outb1_ref):
        dinv = lax.rsqrt(deg_ref[...] + 1.0)
        mcat = jnp.concatenate([m_ref[0], m_ref[1]], axis=1)
        x = dinv * (mcat + hp_ref[...]) + b_ref[...]
        x = jnp.maximum(x, 0.0)
        h = lax.dot_general(x, w_ref[...], (((1,), (1,)), ((), ())),
                            preferred_element_type=F32)
        out_ref[...] = h * dinv
        hpp = (lax.dot_general(x, wp_ref[...], (((1,), (1,)), ((), ())),
                               preferred_element_type=F32) * dinv)
        outb0_ref[...] = hpp[:, :d // 2].astype(BF16)
        outb1_ref[...] = hpp[:, d // 2:].astype(BF16)

    return pl.pallas_call(
        body,
        grid=(steps,),
        in_specs=[
            pl.BlockSpec((2, blk, d // 2), lambda i: (0, i, 0)),
            pl.BlockSpec((blk, d), lambda i: (i, 0)),
            pl.BlockSpec((blk, d), lambda i: (i, 0)),
            pl.BlockSpec((1, d), lambda i: (0, 0)),
            pl.BlockSpec((d, d), lambda i: (0, 0)),
            pl.BlockSpec((d, d), lambda i: (0, 0)),
        ],
        out_specs=[pl.BlockSpec((blk, d), lambda i: (i, 0)),
                   pl.BlockSpec((blk, d // 2), lambda i: (i, 0)),
                   pl.BlockSpec((blk, d // 2), lambda i: (i, 0))],
        out_shape=[jax.ShapeDtypeStruct((np_, d), F32),
                   jax.ShapeDtypeStruct((np_, d // 2), BF16),
                   jax.ShapeDtypeStruct((np_, d // 2), BF16)],
    )(m, hp, degb, b_prev, w, wp)


def _tc_epilog(m, hp, degb, b3, batch_f, xmean, wimg_t, w_i, b_i, wc_img,
               wc_g, b_c, b_img, np_, d, blk, inv_hw):
    steps = np_ // blk

    def body(m_ref, hp_ref, deg_ref, b3_ref, bat_ref, xmean_ref, wt_ref,
             wi_ref, bi_ref, wci_ref, wcg_ref, bc_ref, bimg_ref,
             oimg_ref, ograph_ref, sums, cnt):
        i = pl.program_id(0)

        @pl.when(i == 0)
        def _():
            sums[...] = jnp.zeros_like(sums)
            cnt[...] = jnp.zeros_like(cnt)

        dinv = lax.rsqrt(deg_ref[...] + 1.0)
        mcat = jnp.concatenate([m_ref[0], m_ref[1]], axis=1)
        xg = dinv * (mcat + hp_ref[...]) + b3_ref[...]
        ioc = lax.broadcasted_iota(I32, (32, blk), 0).astype(F32)
        oht = (ioc == bat_ref[0]).astype(F32)
        sums[...] += lax.dot_general(oht, xg, (((1,), (0,)), ((), ())),
                                     preferred_element_type=F32)
        cnt[...] += lax.dot_general(oht, jnp.ones((blk, d), F32),
                                    (((1,), (0,)), ((), ())),
                                    preferred_element_type=F32)

        @pl.when(i == steps - 1)
        def _():
            pooled = sums[...] / jnp.maximum(cnt[...], 1.0)
            meanrow = lax.dot_general(jnp.ones((1, 128), F32), xmean_ref[...],
                                      (((1,), (1,)), ((), ())),
                                      preferred_element_type=F32) * inv_hw
            rowi = lax.broadcasted_iota(I32, (32, 96), 0)
            coli = lax.broadcasted_iota(I32, (32, 96), 1)
            msk = (coli // 3 == rowi).astype(F32)
            g = msk * meanrow
            wie = lax.dot_general(wi_ref[...], wt_ref[...],
                                  (((1,), (0,)), ((), ())),
                                  preferred_element_type=F32)
            wce = lax.dot_general(wci_ref[...], wt_ref[...],
                                  (((1,), (0,)), ((), ())),
                                  preferred_element_type=F32)
            bi_x = lax.dot_general(bimg_ref[...], wi_ref[...],
                                   (((1,), (1,)), ((), ())),
                                   preferred_element_type=F32)
            bc_x = lax.dot_general(bimg_ref[...], wci_ref[...],
                                   (((1,), (1,)), ((), ())),
                                   preferred_element_type=F32)
            oimg = lax.dot_general(g, wie, (((1,), (1,)), ((), ())),
                                   preferred_element_type=F32) + bi_ref[...] + bi_x
            oimg_ref[...] = oimg * lax.rsqrt(
                jnp.sum(oimg * oimg, axis=1, keepdims=True))
            og = (lax.dot_general(g, wce, (((1,), (1,)), ((), ())),
                                  preferred_element_type=F32)
                  + lax.dot_general(pooled, wcg_ref[...],
                                    (((1,), (1,)), ((), ())),
                                    preferred_element_type=F32)
                  + bc_ref[...] + bc_x)
            ograph_ref[...] = og * lax.rsqrt(
                jnp.sum(og * og, axis=1, keepdims=True))

    return pl.pallas_call(
        body,
        grid=(steps,),
        in_specs=[
            pl.BlockSpec((2, blk, d // 2), lambda i: (0, i, 0)),
            pl.BlockSpec((blk, d), lambda i: (i, 0)),
            pl.BlockSpec((blk, d), lambda i: (i, 0)),
            pl.BlockSpec((1, d), lambda i: (0, 0)),
            pl.BlockSpec((1, 1, blk), lambda i: (i, 0, 0)),
            pl.BlockSpec((96, 128), lambda i: (0, 0)),
            pl.BlockSpec((4096, 96), lambda i: (0, 0)),
            pl.BlockSpec((d, 4096), lambda i: (0, 0)),
            pl.BlockSpec((1, d), lambda i: (0, 0)),
            pl.BlockSpec((d, 4096), lambda i: (0, 0)),
            pl.BlockSpec((d, d), lambda i: (0, 0)),
            pl.BlockSpec((1, d), lambda i: (0, 0)),
            pl.BlockSpec((1, 4096), lambda i: (0, 0)),
        ],
        out_specs=[
            pl.BlockSpec((32, d), lambda i: (0, 0)),
            pl.BlockSpec((32, d), lambda i: (0, 0)),
        ],
        out_shape=[
            jax.ShapeDtypeStruct((32, d), F32),
            jax.ShapeDtypeStruct((32, d), F32),
        ],
        scratch_shapes=[pltpu.VMEM((32, d), F32), pltpu.VMEM((32, d), F32)],
    )(m, hp, degb, b3, batch_f, xmean, wimg_t, w_i, b_i, wc_img, wc_g, b_c,
      b_img)


def kernel(images, x_nodes, edge_index, edge_attr, batch, node_table,
           W_img, b_img, W1, b1, W2, b2, W3, b3, W_i, b_i, W_c, b_c):
    n = x_nodes.shape[0]
    e = edge_attr.shape[0]
    d = node_table.shape[1]
    bsz, chans, ih, iw = images.shape
    img_dim = W_img.shape[0]

    np_ = ((n + 2047) // 2048) * 2048
    blk = np_ // 10 if np_ % 10 == 0 else np_ // 8
    ep = ((e + 2 * NW * CHUNK - 1) // (2 * NW * CHUNK)) * (2 * NW * CHUNK)
    ce = ep // (NW * CHUNK)

    src = jnp.pad(edge_index[0].astype(I32), (0, ep - e)).reshape(NW, ce, CHUNK)
    dst = jnp.pad(edge_index[1].astype(I32), (0, ep - e)).reshape(NW, ce, CHUNK)
    w = jnp.pad(edge_attr.astype(F32), (0, ep - e)).reshape(NW, ce, CHUNK)
    xn_f = jnp.pad(x_nodes.astype(F32), (0, np_ - n),
                   constant_values=-1.0).reshape(np_ // blk, 1, blk)
    batch_f = jnp.pad(batch.astype(F32), (0, np_ - n),
                      constant_values=-1.0).reshape(np_ // blk, 1, blk)
    table_p = jnp.pad(node_table.astype(F32),
                      ((0, 32 - node_table.shape[0]), (0, 0)))
    images_r = images.reshape(bsz * chans, ih * iw)
    wimg_t = jnp.concatenate([W_img] * 32, axis=1)
    wc_img = W_c[:, :img_dim]
    wc_g = W_c[:, img_dim:]

    src16 = src.reshape(16, 2 * ce, CHUNK)
    dst16 = dst.reshape(16, 2 * ce, CHUNK)
    w16 = w.reshape(16, 2 * ce, CHUNK)

    degp = _sc_degree(dst, w, np_, ce)
    degb = jnp.broadcast_to((degp[0] + degp[1])[:, None], (np_, d))

    xmean = _tc_image_mean(images_r)

    perm = []
    for k in range(d // 2):
        c, kk = k // 32, k % 32
        j, t = kk // 16, kk % 16
        perm += [64 * c + 32 * j + t, 64 * c + 32 * j + 16 + t]
    perm = jnp.asarray(perm, I32)
    W1p, W2p, W3p = W1[perm], W2[perm], W3[perm]

    h1, ha1, hc1 = _tc_embed_conv1(xn_f, table_p, degb, W1, W1p, np_, d, blk)
    m1 = _sc_messages(ha1, hc1, src16, dst16, w16, np_, d, 2 * ce)
    h2, ha2, hc2 = _tc_conv_next(m1, h1, degb, b1.reshape(1, d), W2, W2p, np_, d, blk)
    m2 = _sc_messages(ha2, hc2, src16, dst16, w16, np_, d, 2 * ce)
    h3, ha3, hc3 = _tc_conv_next(m2, h2, degb, b2.reshape(1, d), W3, W3p, np_, d, blk)
    m3 = _sc_messages(ha3, hc3, src16, dst16, w16, np_, d, 2 * ce)

    out_images, out_graphs = _tc_epilog(
        m3, h3, degb, b3.reshape(1, d), batch_f, xmean, wimg_t, W_i,
        b_i.reshape(1, d), wc_img, wc_g, b_c.reshape(1, d),
        b_img.reshape(1, img_dim), np_, d, blk, 1.0 / (ih * iw))
    return (out_images, out_graphs)


# speedup vs baseline: 1.1235x; 1.1235x over previous
import functools

import jax
import jax.numpy as jnp
from jax import lax
from jax.experimental import pallas as pl
from jax.experimental.pallas import tpu as pltpu
from jax.experimental.pallas import tpu_sc as plsc

F32 = jnp.float32
BF16 = jnp.bfloat16
I32 = jnp.int32

NW = 32
LANE = 16
CHUNK = 128


def _sc_mesh():
    return plsc.VectorSubcoreMesh(core_axis_name="c", subcore_axis_name="s")


def _sc_degree(dst_r, w_r, np_, ce):
    rows_per_tile = np_ // 16

    def body(dst_hbm, w_hbm, out_hbm, dstv, wv, zbuf, acc, _sem):
        cid = lax.axis_index("c")
        sid = lax.axis_index("s")
        wid = sid * 2 + cid
        pltpu.sync_copy(dst_hbm.at[wid], dstv)
        pltpu.sync_copy(w_hbm.at[wid], wv)

        @pl.loop(0, CHUNK // LANE)
        def _zero(j):
            zbuf[pl.ds(j * LANE, LANE)] = jnp.zeros((LANE,), F32)

        @pl.loop(0, rows_per_tile // CHUNK)
        def _zacc(j):
            pltpu.sync_copy(zbuf, acc.at[pl.ds(sid * rows_per_tile + j * CHUNK, CHUNK)])

        plsc.subcore_barrier()

        @pl.loop(0, ce)
        def _scatter(c):
            pltpu.sync_copy(wv.at[c], acc.at[dstv.at[c]], add=True)

        plsc.subcore_barrier()
        pltpu.sync_copy(acc.at[pl.ds(sid * rows_per_tile, rows_per_tile)],
                        out_hbm.at[cid, pl.ds(sid * rows_per_tile, rows_per_tile)])

    k = pl.kernel(
        body,
        out_type=jax.ShapeDtypeStruct((2, np_), F32),
        mesh=_sc_mesh(),
        compiler_params=pltpu.CompilerParams(needs_layout_passes=False),
        scratch_types=[
            pltpu.VMEM((ce, CHUNK), I32),
            pltpu.VMEM((ce, CHUNK), F32),
            pltpu.VMEM((CHUNK,), F32),
            pltpu.VMEM_SHARED((np_,), F32),
            pltpu.SemaphoreType.DMA,
        ],
    )
    return k(dst_r, w_r)


def _sc_messages(hb, src_r, dst_r, w_r, np_, d, ce):
    rows_per_tile = np_ // 16
    hb = lax.bitcast_convert_type(hb.reshape(np_, d // 2, 2), F32)

    def body(h_hbm, src_hbm, dst_hbm, w_hbm, out_hbm, srcv,
             rows0, rows1, srows, dstb0, dstb1, wb0, wb1,
             gsem0, gsem1, dsem0, dsem1, acc):
        cid = lax.axis_index("c")
        sid = lax.axis_index("s")
        wid = sid * 2 + cid
        pltpu.sync_copy(src_hbm.at[wid], srcv)

        @pl.loop(0, CHUNK)
        def _zrows(g):
            for j in range(d // LANE):
                srows[g, pl.ds(j * LANE, LANE)] = jnp.zeros((LANE,), F32)

        @pl.loop(0, rows_per_tile // CHUNK)
        def _zacc(j):
            pltpu.sync_copy(srows, acc.at[pl.ds(sid * rows_per_tile + j * CHUNK, CHUNK)])

        plsc.subcore_barrier()

        def issue(c, rows, dstb, wb, gsem, dsem):
            pltpu.async_copy(h_hbm.at[srcv.at[c]], rows, gsem)
            pltpu.async_copy(dst_hbm.at[wid, c], dstb, dsem)
            pltpu.async_copy(w_hbm.at[wid, c], wb, dsem)

        def drain(c, rows, dstb, wb, gsem, dsem):
            pltpu.make_async_copy(h_hbm.at[srcv.at[c]], rows, gsem).wait()
            pltpu.make_async_copy(dst_hbm.at[wid, c], dstb, dsem).wait()
            pltpu.make_async_copy(w_hbm.at[wid, c], wb, dsem).wait()

        def scale(rows, wb):
            @pl.loop(0, CHUNK)
            def _scale(g):
                wbr = plsc.load_gather(wb, [jnp.full((LANE,), g, I32)])
                for j in range(d // (2 * LANE)):
                    ab = plsc.bitcast(rows[g, pl.ds(LANE * j, LANE)], BF16)
                    a, b = plsc.unpack(ab, format=plsc.PackFormat.INTERLEAVED)
                    srows[g, pl.ds(2 * LANE * j, LANE)] = a * wbr
                    srows[g, pl.ds(2 * LANE * j + LANE, LANE)] = b * wbr

        issue(0, rows0, dstb0, wb0, gsem0, dsem0)
        issue(1, rows1, dstb1, wb1, gsem1, dsem1)

        @pl.loop(0, ce // 2)
        def _edge_pair(k):
            c0 = 2 * k
            drain(c0, rows0, dstb0, wb0, gsem0, dsem0)
            scale(rows0, wb0)
            pltpu.sync_copy(srows, acc.at[dstb0], add=True)

            @pl.when(k < ce // 2 - 1)
            def _():
                issue(c0 + 2, rows0, dstb0, wb0, gsem0, dsem0)

            drain(c0 + 1, rows1, dstb1, wb1, gsem1, dsem1)
            scale(rows1, wb1)
            pltpu.sync_copy(srows, acc.at[dstb1], add=True)

            @pl.when(k < ce // 2 - 1)
            def _():
                issue(c0 + 3, rows1, dstb1, wb1, gsem1, dsem1)

        plsc.subcore_barrier()
        pltpu.sync_copy(acc.at[pl.ds(sid * rows_per_tile, rows_per_tile)],
                        out_hbm.at[cid, pl.ds(sid * rows_per_tile, rows_per_tile)])

    k = pl.kernel(
        body,
        out_type=jax.ShapeDtypeStruct((2, np_, d), F32),
        mesh=_sc_mesh(),
        compiler_params=pltpu.CompilerParams(needs_layout_passes=False,
                                             use_tc_tiling_on_sc=False),
        scratch_types=[
            pltpu.VMEM((ce, CHUNK), I32),
            pltpu.VMEM((CHUNK, d // 2), F32),
            pltpu.VMEM((CHUNK, d // 2), F32),
            pltpu.VMEM((CHUNK, d), F32),
            pltpu.VMEM((CHUNK,), I32),
            pltpu.VMEM((CHUNK,), I32),
            pltpu.VMEM((CHUNK,), F32),
            pltpu.VMEM((CHUNK,), F32),
            pltpu.SemaphoreType.DMA,
            pltpu.SemaphoreType.DMA,
            pltpu.SemaphoreType.DMA,
            pltpu.SemaphoreType.DMA,
            pltpu.VMEM_SHARED((np_, d), F32),
        ],
    )
    return k(hb, src_r, dst_r, w_r)


def _tc_image_mean(images_r):
    rows, cols = images_r.shape
    blk = 1024
    steps = cols // blk

    def body(x_ref, out_ref, acc):
        i = pl.program_id(0)

        @pl.when(i == 0)
        def _():
            acc[...] = jnp.zeros_like(acc)

        x = x_ref[...]
        s = jnp.zeros((rows, 128), F32)
        for j in range(blk // 128):
            s = s + x[:, j * 128:(j + 1) * 128]
        acc[...] += s

        @pl.when(i == steps - 1)
        def _():
            out_ref[...] = acc[...]

    return pl.pallas_call(
        body,
        grid=(steps,),
        in_specs=[pl.BlockSpec((rows, blk), lambda i: (0, i))],
        out_specs=pl.BlockSpec((rows, 128), lambda i: (0, 0)),
        out_shape=jax.ShapeDtypeStruct((rows, 128), F32),
        scratch_shapes=[pltpu.VMEM((rows, 128), F32)],
    )(images_r)


def _tc_embed_conv1(xn_f, table_p, degb, w1, w1p, np_, d, blk):
    steps = np_ // blk

    def body(xn_ref, tbl_ref, deg_ref, w_ref, wp_ref, out_ref, outb_ref):
        xn_row = xn_ref[0]
        ioc = lax.broadcasted_iota(I32, (32, blk), 0).astype(F32)
        oht = (ioc == xn_row).astype(F32)
        xg = lax.dot_general(oht, tbl_ref[...],
                             (((0,), (0,)), ((), ())),
                             preferred_element_type=F32)
        dinv = lax.rsqrt(deg_ref[...] + 1.0)
        h = lax.dot_general(xg, w_ref[...], (((1,), (1,)), ((), ())),
                            preferred_element_type=F32)
        out_ref[...] = h * dinv
        hp = lax.dot_general(xg, wp_ref[...], (((1,), (1,)), ((), ())),
                             preferred_element_type=F32)
        outb_ref[...] = (hp * dinv).astype(BF16)

    return pl.pallas_call(
        body,
        grid=(steps,),
        in_specs=[
            pl.BlockSpec((1, 1, blk), lambda i: (i, 0, 0)),
            pl.BlockSpec((32, d), lambda i: (0, 0)),
            pl.BlockSpec((blk, d), lambda i: (i, 0)),
            pl.BlockSpec((d, d), lambda i: (0, 0)),
            pl.BlockSpec((d, d), lambda i: (0, 0)),
        ],
        out_specs=[pl.BlockSpec((blk, d), lambda i: (i, 0)),
                   pl.BlockSpec((blk, d), lambda i: (i, 0))],
        out_shape=[jax.ShapeDtypeStruct((np_, d), F32),
                   jax.ShapeDtypeStruct((np_, d), BF16)],
    )(xn_f, table_p, degb, w1, w1p)


def _tc_conv_next(m, hp, degb, b_prev, w, wp, np_, d, blk):
    steps = np_ // blk

    def body(m_ref, hp_ref, deg_ref, b_ref, w_ref, wp_ref, out_ref, outb_ref):
        dinv = lax.rsqrt(deg_ref[...] + 1.0)
        x = dinv * (m_ref[0] + m_ref[1] + hp_ref[...]) + b_ref[...]
        x = jnp.maximum(x, 0.0)
        h = lax.dot_general(x, w_ref[...], (((1,), (1,)), ((), ())),
                            preferred_element_type=F32)
        out_ref[...] = h * dinv
        hpp = lax.dot_general(x, wp_ref[...], (((1,), (1,)), ((), ())),
                              preferred_element_type=F32)
        outb_ref[...] = (hpp * dinv).astype(BF16)

    return pl.pallas_call(
        body,
        grid=(steps,),
        in_specs=[
            pl.BlockSpec((2, blk, d), lambda i: (0, i, 0)),
            pl.BlockSpec((blk, d), lambda i: (i, 0)),
            pl.BlockSpec((blk, d), lambda i: (i, 0)),
            pl.BlockSpec((1, d), lambda i: (0, 0)),
            pl.BlockSpec((d, d), lambda i: (0, 0)),
            pl.BlockSpec((d, d), lambda i: (0, 0)),
        ],
        out_specs=[pl.BlockSpec((blk, d), lambda i: (i, 0)),
                   pl.BlockSpec((blk, d), lambda i: (i, 0))],
        out_shape=[jax.ShapeDtypeStruct((np_, d), F32),
                   jax.ShapeDtypeStruct((np_, d), BF16)],
    )(m, hp, degb, b_prev, w, wp)


def _tc_epilog(m, hp, degb, b3, batch_f, xmean, wimg_t, w_i, b_i, wc_img,
               wc_g, b_c, b_img, np_, d, blk, inv_hw):
    steps = np_ // blk

    def body(m_ref, hp_ref, deg_ref, b3_ref, bat_ref, xmean_ref, wt_ref,
             wi_ref, bi_ref, wci_ref, wcg_ref, bc_ref, bimg_ref,
             oimg_ref, ograph_ref, sums, cnt):
        i = pl.program_id(0)

        @pl.when(i == 0)
        def _():
            sums[...] = jnp.zeros_like(sums)
            cnt[...] = jnp.zeros_like(cnt)

        dinv = lax.rsqrt(deg_ref[...] + 1.0)
        xg = dinv * (m_ref[0] + m_ref[1] + hp_ref[...]) + b3_ref[...]
        ioc = lax.broadcasted_iota(I32, (32, blk), 0).astype(F32)
        oht = (ioc == bat_ref[0]).astype(F32)
        sums[...] += lax.dot_general(oht, xg, (((1,), (0,)), ((), ())),
                                     preferred_element_type=F32)
        cnt[...] += lax.dot_general(oht, jnp.ones((blk, d), F32),
                                    (((1,), (0,)), ((), ())),
                                    preferred_element_type=F32)

        @pl.when(i == steps - 1)
        def _():
            pooled = sums[...] / jnp.maximum(cnt[...], 1.0)
            meanrow = lax.dot_general(jnp.ones((1, 128), F32), xmean_ref[...],
                                      (((1,), (1,)), ((), ())),
                                      preferred_element_type=F32) * inv_hw
            rowi = lax.broadcasted_iota(I32, (32, 96), 0)
            coli = lax.broadcasted_iota(I32, (32, 96), 1)
            msk = (coli // 3 == rowi).astype(F32)
            g = msk * meanrow
            wie = lax.dot_general(wi_ref[...], wt_ref[...],
                                  (((1,), (0,)), ((), ())),
                                  preferred_element_type=F32)
            wce = lax.dot_general(wci_ref[...], wt_ref[...],
                                  (((1,), (0,)), ((), ())),
                                  preferred_element_type=F32)
            bi_x = lax.dot_general(bimg_ref[...], wi_ref[...],
                                   (((1,), (1,)), ((), ())),
                                   preferred_element_type=F32)
            bc_x = lax.dot_general(bimg_ref[...], wci_ref[...],
                                   (((1,), (1,)), ((), ())),
                                   preferred_element_type=F32)
            oimg = lax.dot_general(g, wie, (((1,), (1,)), ((), ())),
                                   preferred_element_type=F32) + bi_ref[...] + bi_x
            oimg_ref[...] = oimg * lax.rsqrt(
                jnp.sum(oimg * oimg, axis=1, keepdims=True))
            og = (lax.dot_general(g, wce, (((1,), (1,)), ((), ())),
                                  preferred_element_type=F32)
                  + lax.dot_general(pooled, wcg_ref[...],
                                    (((1,), (1,)), ((), ())),
                                    preferred_element_type=F32)
                  + bc_ref[...] + bc_x)
            ograph_ref[...] = og * lax.rsqrt(
                jnp.sum(og * og, axis=1, keepdims=True))

    return pl.pallas_call(
        body,
        grid=(steps,),
        in_specs=[
            pl.BlockSpec((2, blk, d), lambda i: (0, i, 0)),
            pl.BlockSpec((blk, d), lambda i: (i, 0)),
            pl.BlockSpec((blk, d), lambda i: (i, 0)),
            pl.BlockSpec((1, d), lambda i: (0, 0)),
            pl.BlockSpec((1, 1, blk), lambda i: (i, 0, 0)),
            pl.BlockSpec((96, 128), lambda i: (0, 0)),
            pl.BlockSpec((4096, 96), lambda i: (0, 0)),
            pl.BlockSpec((d, 4096), lambda i: (0, 0)),
            pl.BlockSpec((1, d), lambda i: (0, 0)),
            pl.BlockSpec((d, 4096), lambda i: (0, 0)),
            pl.BlockSpec((d, d), lambda i: (0, 0)),
            pl.BlockSpec((1, d), lambda i: (0, 0)),
            pl.BlockSpec((1, 4096), lambda i: (0, 0)),
        ],
        out_specs=[
            pl.BlockSpec((32, d), lambda i: (0, 0)),
            pl.BlockSpec((32, d), lambda i: (0, 0)),
        ],
        out_shape=[
            jax.ShapeDtypeStruct((32, d), F32),
            jax.ShapeDtypeStruct((32, d), F32),
        ],
        scratch_shapes=[pltpu.VMEM((32, d), F32), pltpu.VMEM((32, d), F32)],
    )(m, hp, degb, b3, batch_f, xmean, wimg_t, w_i, b_i, wc_img, wc_g, b_c,
      b_img)


def kernel(images, x_nodes, edge_index, edge_attr, batch, node_table,
           W_img, b_img, W1, b1, W2, b2, W3, b3, W_i, b_i, W_c, b_c):
    n = x_nodes.shape[0]
    e = edge_attr.shape[0]
    d = node_table.shape[1]
    bsz, chans, ih, iw = images.shape
    img_dim = W_img.shape[0]

    np_ = ((n + 2047) // 2048) * 2048
    blk = np_ // 10 if np_ % 10 == 0 else np_ // 8
    ep = ((e + 2 * NW * CHUNK - 1) // (2 * NW * CHUNK)) * (2 * NW * CHUNK)
    ce = ep // (NW * CHUNK)

    src = jnp.pad(edge_index[0].astype(I32), (0, ep - e)).reshape(NW, ce, CHUNK)
    dst = jnp.pad(edge_index[1].astype(I32), (0, ep - e)).reshape(NW, ce, CHUNK)
    w = jnp.pad(edge_attr.astype(F32), (0, ep - e)).reshape(NW, ce, CHUNK)
    xn_f = jnp.pad(x_nodes.astype(F32), (0, np_ - n),
                   constant_values=-1.0).reshape(np_ // blk, 1, blk)
    batch_f = jnp.pad(batch.astype(F32), (0, np_ - n),
                      constant_values=-1.0).reshape(np_ // blk, 1, blk)
    table_p = jnp.pad(node_table.astype(F32),
                      ((0, 32 - node_table.shape[0]), (0, 0)))
    images_r = images.reshape(bsz * chans, ih * iw)
    wimg_t = jnp.concatenate([W_img] * 32, axis=1)
    wc_img = W_c[:, :img_dim]
    wc_g = W_c[:, img_dim:]

    degp = _sc_degree(dst, w, np_, ce)
    degb = jnp.broadcast_to((degp[0] + degp[1])[:, None], (np_, d))

    xmean = _tc_image_mean(images_r)

    perm = jnp.asarray([32 * j + (k // 2 if k % 2 == 0 else 16 + (k - 1) // 2)
                        for j in range(d // 32) for k in range(32)], I32)
    W1p, W2p, W3p = W1[perm], W2[perm], W3[perm]

    h1, hb1 = _tc_embed_conv1(xn_f, table_p, degb, W1, W1p, np_, d, blk)
    m1 = _sc_messages(hb1, src, dst, w, np_, d, ce)
    h2, hb2 = _tc_conv_next(m1, h1, degb, b1.reshape(1, d), W2, W2p, np_, d, blk)
    m2 = _sc_messages(hb2, src, dst, w, np_, d, ce)
    h3, hb3 = _tc_conv_next(m2, h2, degb, b2.reshape(1, d), W3, W3p, np_, d, blk)
    m3 = _sc_messages(hb3, src, dst, w, np_, d, ce)

    out_images, out_graphs = _tc_epilog(
        m3, h3, degb, b3.reshape(1, d), batch_f, xmean, wimg_t, W_i,
        b_i.reshape(1, d), wc_img, wc_g, b_c.reshape(1, d),
        b_img.reshape(1, img_dim), np_, d, blk, 1.0 / (ih * iw))
    return (out_images, out_graphs)
